# baseline (speedup 1.0000x reference)
import functools
import math

import jax
import jax.numpy as jnp
from jax import lax
from jax.experimental import pallas as pl
from jax.experimental.pallas import tpu as pltpu
from jax.experimental.pallas import tpu_sc as plsc

N = 10000
E = 160000
D = 128
RB = 8
HID = 64
S = 10
AVG = 16.0

NODE_BLK = 2000
EDGE_BLK = 4480

NTEC = 16
CHUNK = 48
CPI = 10
E_P = 161280
EPT_P = E_P // NTEC
ITERS = EPT_P // (CPI * CHUNK)
HALF = N // 2
GARBAGE = HALF
AGG_ROWS = 5120
ROWS_PER_TEC = AGG_ROWS // NTEC
CH = D // 2


def _node_kernel(x_ref, oh_ref, wup_ref, wskip_ref, feats_ref, sc_ref):
    x = x_ref[...]
    scale = 1.0 / math.sqrt(float(D))
    f = jnp.dot(x, wup_ref[...], preferred_element_type=jnp.float32) * scale
    feats_ref[0] = f[:, :CH]
    feats_ref[1] = f[:, CH:]
    acc = jnp.zeros_like(x)
    for s in range(S):
        xs = x * oh_ref[:, s][:, None]
        acc = acc + jnp.dot(xs, wskip_ref[s], preferred_element_type=jnp.float32)
    sc_ref[...] = acc * scale


def _silu(x):
    return x * jax.nn.sigmoid(x)


def _mlp_kernel(rb_ref, w1_ref, w2_ref, w3_ref, rw_ref):
    h = _silu(jnp.dot(rb_ref[...], w1_ref[...], preferred_element_type=jnp.float32)
              * (1.0 / math.sqrt(float(RB))))
    h = _silu(jnp.dot(h, w2_ref[...], preferred_element_type=jnp.float32)
              * (1.0 / math.sqrt(float(HID))))
    rw = (jnp.dot(h, w3_ref[...], preferred_element_type=jnp.float32)
          * (1.0 / math.sqrt(float(HID))))
    rw_ref[0, :, :CH] = rw[:, 0:CH]
    rw_ref[0, :, CH:] = rw[:, D:D + CH]
    rw_ref[1, :, :CH] = rw[:, CH:D]
    rw_ref[1, :, CH:] = rw[:, D + CH:2 * D]


def _edge_sc_kernel(sh_hbm, send_hbm, recv_hbm, feats_hbm, rw_hbm, agg_hbm,
                    send_v, recv_v, sh_v, idx_v, msg_v, rw_v, pay_v, zero_v,
                    agg_sh, meta_sem, in_sem0, in_sem1, sc_sem0, sc_sem1):
    core = lax.axis_index("c")
    sid = lax.axis_index("s")
    in_sems = (in_sem0, in_sem1)
    sc_sems = (sc_sem0, sc_sem1)

    def _zrow(r, _):
        for v in range(16):
            zero_v[r, pl.ds(v * 16, 16)] = jnp.zeros((16,), jnp.float32)
        return 0
    lax.fori_loop(0, 8, _zrow, 0)
    plsc.subcore_barrier()

    for p in range(2):
        lo = p * HALF

        zbase = sid * ROWS_PER_TEC

        def _zcopy(i, _):
            pltpu.sync_copy(zero_v, agg_sh.at[pl.ds(zbase + i * 8, 8)])
            return 0
        lax.fori_loop(0, ROWS_PER_TEC // 8, _zcopy, 0)
        plsc.subcore_barrier()

        def iter_body(t, _):
            ibase = sid * EPT_P + t * (CPI * CHUNK)
            h1 = pltpu.async_copy(send_hbm.at[pl.ds(ibase, CPI * CHUNK)],
                                  send_v, meta_sem)
            h2 = pltpu.async_copy(recv_hbm.at[pl.ds(ibase, CPI * CHUNK)],
                                  recv_v, meta_sem)
            h3 = pltpu.async_copy(sh_hbm.at[:, pl.ds(ibase, CPI * CHUNK)],
                                  sh_v, meta_sem)
            h1.wait()
            h2.wait()
            h3.wait()

            def fire_inputs(i):
                b = i % 2
                cb = ibase + i * CHUNK
                hg = pltpu.async_copy(
                    feats_hbm.at[core].at[send_v.at[pl.ds(i * CHUNK, CHUNK)]],
                    msg_v.at[b], in_sems[b])
                hr = pltpu.async_copy(rw_hbm.at[core, pl.ds(cb, CHUNK), :],
                                      rw_v.at[b], in_sems[b])
                return hg, hr

            h_in = [fire_inputs(0), fire_inputs(1)] + [None] * (CPI - 2)
            h_sc = [None] * CPI
            row_ids = [jnp.full((16,), j, jnp.int32) for j in range(4)]
            for i in range(CPI):
                b = i % 2
                hg, hr = h_in[i]
                hg.wait()
                hr.wait()
                if i >= 2:
                    h_sc[i - 2].wait()

                def idx_body(g, _):
                    rv = recv_v[pl.ds(i * CHUNK + g * 16, 16)]
                    valid = (rv >= lo) & (rv < lo + HALF)
                    idx_v[b, pl.ds(g * 16, 16)] = jnp.where(valid, rv - lo,
                                                            GARBAGE)
                    return 0
                lax.fori_loop(0, CHUNK // 16, idx_body, 0)

                def edge_body(e, _):
                    es = jnp.full((16,), i * CHUNK + e, jnp.int32)
                    r = plsc.load_gather(recv_v, [es])[0]

                    @pl.when((r >= lo) & (r < lo + HALF))
                    def _():
                        s0 = plsc.load_gather(sh_v, [row_ids[0], es])
                        s1 = plsc.load_gather(sh_v, [row_ids[1], es])
                        s2 = plsc.load_gather(sh_v, [row_ids[2], es])
                        s3 = plsc.load_gather(sh_v, [row_ids[3], es])
                        for v in range(CH // 16):
                            m = msg_v[b, e, pl.ds(v * 16, 16)]
                            w0 = rw_v[b, e, pl.ds(v * 16, 16)]
                            w1 = rw_v[b, e, pl.ds(CH + v * 16, 16)]
                            p0 = m * w0
                            p1 = m * w1
                            pay_v[b, e, pl.ds(v * 16, 16)] = p0 * s0
                            pay_v[b, e, pl.ds(CH + v * 16, 16)] = p1 * s1
                            pay_v[b, e, pl.ds(2 * CH + v * 16, 16)] = p1 * s2
                            pay_v[b, e, pl.ds(3 * CH + v * 16, 16)] = p1 * s3
                    return 0
                lax.fori_loop(0, CHUNK, edge_body, 0)

                h_sc[i] = pltpu.async_copy(pay_v.at[b],
                                           agg_sh.at[idx_v.at[b]],
                                           sc_sems[b], add=True)
                if i + 2 < CPI:
                    h_in[i + 2] = fire_inputs(i + 2)
            h_sc[CPI - 2].wait()
            h_sc[CPI - 1].wait()
            return 0
        lax.fori_loop(0, ITERS, iter_body, 0)
        plsc.subcore_barrier()

        obase = sid * 312
        for off in (0, 104, 208):
            pltpu.sync_copy(agg_sh.at[pl.ds(obase + off, 104)],
                            agg_hbm.at[core, pl.ds(lo + obase + off, 104)])

        @pl.when(sid == NTEC - 1)
        def _():
            pltpu.sync_copy(agg_sh.at[pl.ds(4992, 8)],
                            agg_hbm.at[core, pl.ds(lo + 4992, 8)])
        plsc.subcore_barrier()


def _mix_kernel(agg_ref, w0_ref, w1_ref, out_ref):
    scale = 1.0 / (math.sqrt(float(D)) * AVG)
    a0 = agg_ref[0]
    a1 = agg_ref[1]
    for j in range(4):
        w = w0_ref[...] if j == 0 else w1_ref[...]
        o = (jnp.dot(a0[:, j * CH:(j + 1) * CH], w[:CH, :],
                     preferred_element_type=jnp.float32)
             + jnp.dot(a1[:, j * CH:(j + 1) * CH], w[CH:, :],
                       preferred_element_type=jnp.float32))
        out_ref[j] = o * scale


def kernel(node_feats, species, sh, radial_basis, senders, receivers,
           W_up, W_mlp1, W_mlp2, W_mlp3, W_lin0, W_lin1, W_skip):
    n, d = node_feats.shape

    onehot = jax.nn.one_hot(species, S, dtype=jnp.float32)

    feats_sc, sc = pl.pallas_call(
        _node_kernel,
        grid=(n // NODE_BLK,),
        in_specs=[
            pl.BlockSpec((NODE_BLK, d), lambda i: (i, 0)),
            pl.BlockSpec((NODE_BLK, S), lambda i: (i, 0)),
            pl.BlockSpec((d, d), lambda i: (0, 0)),
            pl.BlockSpec((S, d, d), lambda i: (0, 0, 0)),
        ],
        out_specs=[
            pl.BlockSpec((2, NODE_BLK, CH), lambda i: (0, i, 0)),
            pl.BlockSpec((NODE_BLK, d), lambda i: (i, 0)),
        ],
        out_shape=[
            jax.ShapeDtypeStruct((2, n, CH), jnp.float32),
            jax.ShapeDtypeStruct((n, d), jnp.float32),
        ],
    )(node_feats, onehot, W_up, W_skip)

    pad = E_P - E
    rb_p = jnp.pad(radial_basis, ((0, pad), (0, 0)))
    send_p = jnp.pad(senders, (0, pad))
    recv_p = jnp.pad(receivers, (0, pad), constant_values=N)
    shT_p = jnp.pad(sh.T, ((0, 0), (0, pad)))

    rw_sc = pl.pallas_call(
        _mlp_kernel,
        grid=(E_P // EDGE_BLK,),
        in_specs=[
            pl.BlockSpec((EDGE_BLK, RB), lambda i: (i, 0)),
            pl.BlockSpec((RB, HID), lambda i: (0, 0)),
            pl.BlockSpec((HID, HID), lambda i: (0, 0)),
            pl.BlockSpec((HID, 2 * D), lambda i: (0, 0)),
        ],
        out_specs=pl.BlockSpec((2, EDGE_BLK, d), lambda i: (0, i, 0)),
        out_shape=jax.ShapeDtypeStruct((2, E_P, d), jnp.float32),
    )(rb_p, W_mlp1, W_mlp2, W_mlp3)

    mesh = plsc.VectorSubcoreMesh(core_axis_name="c", subcore_axis_name="s")
    agg = pl.kernel(
        _edge_sc_kernel,
        out_type=jax.ShapeDtypeStruct((2, n, 4 * CH), jnp.float32),
        mesh=mesh,
        scratch_types=[
            pltpu.VMEM((CPI * CHUNK,), jnp.int32),
            pltpu.VMEM((CPI * CHUNK,), jnp.int32),
            pltpu.VMEM((4, CPI * CHUNK), jnp.float32),
            pltpu.VMEM((2, CHUNK), jnp.int32),
            pltpu.VMEM((2, CHUNK, CH), jnp.float32),
            pltpu.VMEM((2, CHUNK, 2 * CH), jnp.float32),
            pltpu.VMEM((2, CHUNK, 4 * CH), jnp.float32),
            pltpu.VMEM((8, 4 * CH), jnp.float32),
            pltpu.VMEM_SHARED((AGG_ROWS, 4 * CH), jnp.float32),
            pltpu.SemaphoreType.DMA,
            pltpu.SemaphoreType.DMA,
            pltpu.SemaphoreType.DMA,
            pltpu.SemaphoreType.DMA,
            pltpu.SemaphoreType.DMA,
        ],
        compiler_params=pltpu.CompilerParams(use_tc_tiling_on_sc=False,
                                             needs_layout_passes=False),
    )(shT_p, send_p, recv_p, feats_sc, rw_sc)

    W_lins = jnp.stack([W_lin0, W_lin1], axis=0)
    out = pl.pallas_call(
        _mix_kernel,
        grid=(n // NODE_BLK,),
        in_specs=[
            pl.BlockSpec((2, NODE_BLK, 4 * CH), lambda i: (0, i, 0)),
            pl.BlockSpec((d, d), lambda i: (0, 0)),
            pl.BlockSpec((d, d), lambda i: (0, 0)),
        ],
        out_specs=pl.BlockSpec((4, NODE_BLK, d), lambda i: (0, i, 0)),
        out_shape=jax.ShapeDtypeStruct((4, n, d), jnp.float32),
    )(agg, W_lin0, W_lin1)

    message = jnp.concatenate(
        [out[0], out[1:4].transpose(1, 2, 0).reshape(n, 3 * d)], axis=-1)
    return (message, sc)


# speedup vs baseline: 1.5230x; 1.5230x over previous
import functools
import math

import jax
import jax.numpy as jnp
from jax import lax
from jax.experimental import pallas as pl
from jax.experimental.pallas import tpu as pltpu
from jax.experimental.pallas import tpu_sc as plsc

N = 10000
E = 160000
D = 128
RB = 8
HID = 64
S = 10
AVG = 16.0

NODE_BLK = 2000
EDGE_BLK = 4480

NTEC = 16
CHUNK = 48
CPI = 10
E_P = 161280
EPT_P = E_P // NTEC
ITERS = EPT_P // (CPI * CHUNK)
HALF = N // 2
GARBAGE = HALF
AGG_ROWS = 5120
ROWS_PER_TEC = AGG_ROWS // NTEC
CH = D // 2


def _node_kernel(x_ref, oh_ref, wup_ref, wskip_ref, feats_ref, sc_ref):
    x = x_ref[...]
    scale = 1.0 / math.sqrt(float(D))
    f = jnp.dot(x, wup_ref[...], preferred_element_type=jnp.float32) * scale
    feats_ref[0] = f[:, :CH]
    feats_ref[1] = f[:, CH:]
    acc = jnp.zeros_like(x)
    for s in range(S):
        xs = x * oh_ref[:, s][:, None]
        acc = acc + jnp.dot(xs, wskip_ref[s], preferred_element_type=jnp.float32)
    sc_ref[...] = acc * scale


def _silu(x):
    return x * jax.nn.sigmoid(x)


def _mlp_kernel(rb_ref, w1_ref, w2_ref, w3_ref, rw_ref):
    h = _silu(jnp.dot(rb_ref[...], w1_ref[...], preferred_element_type=jnp.float32)
              * (1.0 / math.sqrt(float(RB))))
    h = _silu(jnp.dot(h, w2_ref[...], preferred_element_type=jnp.float32)
              * (1.0 / math.sqrt(float(HID))))
    rw = (jnp.dot(h, w3_ref[...], preferred_element_type=jnp.float32)
          * (1.0 / math.sqrt(float(HID))))
    rw_ref[0, :, :CH] = rw[:, 0:CH]
    rw_ref[0, :, CH:] = rw[:, D:D + CH]
    rw_ref[1, :, :CH] = rw[:, CH:D]
    rw_ref[1, :, CH:] = rw[:, D + CH:2 * D]


def _edge_sc_kernel(sh_hbm, send_hbm, recv_hbm, feats_hbm, rw_hbm, agg_hbm,
                    send_v, recv_v, sh_v, idx_v, msg_v, rw_v, pay_v, zero_v,
                    agg_sh, meta_sem, in_sem0, in_sem1, sc_sem0, sc_sem1):
    core = lax.axis_index("c")
    sid = lax.axis_index("s")
    in_sems = (in_sem0, in_sem1)
    sc_sems = (sc_sem0, sc_sem1)

    def _zrow(r, _):
        for v in range(16):
            zero_v[r, pl.ds(v * 16, 16)] = jnp.zeros((16,), jnp.float32)
        return 0
    lax.fori_loop(0, 8, _zrow, 0)
    plsc.subcore_barrier()

    for p in range(2):
        lo = p * HALF

        zbase = sid * ROWS_PER_TEC

        def _zcopy(i, _):
            pltpu.sync_copy(zero_v, agg_sh.at[pl.ds(zbase + i * 8, 8)])
            return 0
        lax.fori_loop(0, ROWS_PER_TEC // 8, _zcopy, 0)
        plsc.subcore_barrier()

        def iter_body(t, _):
            ibase = sid * EPT_P + t * (CPI * CHUNK)
            h1 = pltpu.async_copy(send_hbm.at[pl.ds(ibase, CPI * CHUNK)],
                                  send_v, meta_sem)
            h2 = pltpu.async_copy(recv_hbm.at[pl.ds(ibase, CPI * CHUNK)],
                                  recv_v, meta_sem)
            h3 = pltpu.async_copy(sh_hbm.at[:, pl.ds(ibase, CPI * CHUNK)],
                                  sh_v, meta_sem)
            h1.wait()
            h2.wait()
            h3.wait()

            def fire_inputs(i):
                b = i % 2
                cb = ibase + i * CHUNK
                hg = pltpu.async_copy(
                    feats_hbm.at[core].at[send_v.at[pl.ds(i * CHUNK, CHUNK)]],
                    msg_v.at[b], in_sems[b])
                hr = pltpu.async_copy(rw_hbm.at[core, pl.ds(cb, CHUNK), :],
                                      rw_v.at[b], in_sems[b])
                return hg, hr

            h_in = [fire_inputs(0), fire_inputs(1)] + [None] * (CPI - 2)
            h_sc = [None] * CPI
            row_ids = [jnp.full((16,), j, jnp.int32) for j in range(4)]
            for i in range(CPI):
                b = i % 2
                hg, hr = h_in[i]
                hg.wait()
                hr.wait()
                if i >= 2:
                    h_sc[i - 2].wait()

                @plsc.parallel_loop(0, CHUNK // 16, unroll=3)
                def idx_body(g):
                    rv = recv_v[pl.ds(i * CHUNK + g * 16, 16)]
                    valid = (rv >= lo) & (rv < lo + HALF)
                    idx_v[b, pl.ds(g * 16, 16)] = jnp.where(valid, rv - lo,
                                                            GARBAGE)

                @plsc.parallel_loop(0, CHUNK, unroll=2)
                def edge_body(e):
                    es = jnp.full((16,), i * CHUNK + e, jnp.int32)
                    s0 = plsc.load_gather(sh_v, [row_ids[0], es])
                    s1 = plsc.load_gather(sh_v, [row_ids[1], es])
                    s2 = plsc.load_gather(sh_v, [row_ids[2], es])
                    s3 = plsc.load_gather(sh_v, [row_ids[3], es])
                    for v in range(CH // 16):
                        m = msg_v[b, e, pl.ds(v * 16, 16)]
                        w0 = rw_v[b, e, pl.ds(v * 16, 16)]
                        w1 = rw_v[b, e, pl.ds(CH + v * 16, 16)]
                        p0 = m * w0
                        p1 = m * w1
                        pay_v[b, e, pl.ds(v * 16, 16)] = p0 * s0
                        pay_v[b, e, pl.ds(CH + v * 16, 16)] = p1 * s1
                        pay_v[b, e, pl.ds(2 * CH + v * 16, 16)] = p1 * s2
                        pay_v[b, e, pl.ds(3 * CH + v * 16, 16)] = p1 * s3

                h_sc[i] = pltpu.async_copy(pay_v.at[b],
                                           agg_sh.at[idx_v.at[b]],
                                           sc_sems[b], add=True)
                if i + 2 < CPI:
                    h_in[i + 2] = fire_inputs(i + 2)
            h_sc[CPI - 2].wait()
            h_sc[CPI - 1].wait()
            return 0
        lax.fori_loop(0, ITERS, iter_body, 0)
        plsc.subcore_barrier()

        obase = sid * 312
        for off in (0, 104, 208):
            pltpu.sync_copy(agg_sh.at[pl.ds(obase + off, 104)],
                            agg_hbm.at[core, pl.ds(lo + obase + off, 104)])

        @pl.when(sid == NTEC - 1)
        def _():
            pltpu.sync_copy(agg_sh.at[pl.ds(4992, 8)],
                            agg_hbm.at[core, pl.ds(lo + 4992, 8)])
        plsc.subcore_barrier()


def _mix_kernel(agg_ref, w0_ref, w1_ref, out_ref):
    scale = 1.0 / (math.sqrt(float(D)) * AVG)
    a0 = agg_ref[0]
    a1 = agg_ref[1]
    for j in range(4):
        w = w0_ref[...] if j == 0 else w1_ref[...]
        o = (jnp.dot(a0[:, j * CH:(j + 1) * CH], w[:CH, :],
                     preferred_element_type=jnp.float32)
             + jnp.dot(a1[:, j * CH:(j + 1) * CH], w[CH:, :],
                       preferred_element_type=jnp.float32))
        out_ref[j] = o * scale


def kernel(node_feats, species, sh, radial_basis, senders, receivers,
           W_up, W_mlp1, W_mlp2, W_mlp3, W_lin0, W_lin1, W_skip):
    n, d = node_feats.shape

    onehot = jax.nn.one_hot(species, S, dtype=jnp.float32)

    feats_sc, sc = pl.pallas_call(
        _node_kernel,
        grid=(n // NODE_BLK,),
        in_specs=[
            pl.BlockSpec((NODE_BLK, d), lambda i: (i, 0)),
            pl.BlockSpec((NODE_BLK, S), lambda i: (i, 0)),
            pl.BlockSpec((d, d), lambda i: (0, 0)),
            pl.BlockSpec((S, d, d), lambda i: (0, 0, 0)),
        ],
        out_specs=[
            pl.BlockSpec((2, NODE_BLK, CH), lambda i: (0, i, 0)),
            pl.BlockSpec((NODE_BLK, d), lambda i: (i, 0)),
        ],
        out_shape=[
            jax.ShapeDtypeStruct((2, n, CH), jnp.float32),
            jax.ShapeDtypeStruct((n, d), jnp.float32),
        ],
    )(node_feats, onehot, W_up, W_skip)

    pad = E_P - E
    rb_p = jnp.pad(radial_basis, ((0, pad), (0, 0)))
    send_p = jnp.pad(senders, (0, pad))
    recv_p = jnp.pad(receivers, (0, pad), constant_values=N)
    shT_p = jnp.pad(sh.T, ((0, 0), (0, pad)))

    rw_sc = pl.pallas_call(
        _mlp_kernel,
        grid=(E_P // EDGE_BLK,),
        in_specs=[
            pl.BlockSpec((EDGE_BLK, RB), lambda i: (i, 0)),
            pl.BlockSpec((RB, HID), lambda i: (0, 0)),
            pl.BlockSpec((HID, HID), lambda i: (0, 0)),
            pl.BlockSpec((HID, 2 * D), lambda i: (0, 0)),
        ],
        out_specs=pl.BlockSpec((2, EDGE_BLK, d), lambda i: (0, i, 0)),
        out_shape=jax.ShapeDtypeStruct((2, E_P, d), jnp.float32),
    )(rb_p, W_mlp1, W_mlp2, W_mlp3)

    mesh = plsc.VectorSubcoreMesh(core_axis_name="c", subcore_axis_name="s")
    agg = pl.kernel(
        _edge_sc_kernel,
        out_type=jax.ShapeDtypeStruct((2, n, 4 * CH), jnp.float32),
        mesh=mesh,
        scratch_types=[
            pltpu.VMEM((CPI * CHUNK,), jnp.int32),
            pltpu.VMEM((CPI * CHUNK,), jnp.int32),
            pltpu.VMEM((4, CPI * CHUNK), jnp.float32),
            pltpu.VMEM((2, CHUNK), jnp.int32),
            pltpu.VMEM((2, CHUNK, CH), jnp.float32),
            pltpu.VMEM((2, CHUNK, 2 * CH), jnp.float32),
            pltpu.VMEM((2, CHUNK, 4 * CH), jnp.float32),
            pltpu.VMEM((8, 4 * CH), jnp.float32),
            pltpu.VMEM_SHARED((AGG_ROWS, 4 * CH), jnp.float32),
            pltpu.SemaphoreType.DMA,
            pltpu.SemaphoreType.DMA,
            pltpu.SemaphoreType.DMA,
            pltpu.SemaphoreType.DMA,
            pltpu.SemaphoreType.DMA,
        ],
        compiler_params=pltpu.CompilerParams(use_tc_tiling_on_sc=False,
                                             needs_layout_passes=False),
    )(shT_p, send_p, recv_p, feats_sc, rw_sc)

    W_lins = jnp.stack([W_lin0, W_lin1], axis=0)
    out = pl.pallas_call(
        _mix_kernel,
        grid=(n // NODE_BLK,),
        in_specs=[
            pl.BlockSpec((2, NODE_BLK, 4 * CH), lambda i: (0, i, 0)),
            pl.BlockSpec((d, d), lambda i: (0, 0)),
            pl.BlockSpec((d, d), lambda i: (0, 0)),
        ],
        out_specs=pl.BlockSpec((4, NODE_BLK, d), lambda i: (0, i, 0)),
        out_shape=jax.ShapeDtypeStruct((4, n, d), jnp.float32),
    )(agg, W_lin0, W_lin1)

    message = jnp.concatenate(
        [out[0], out[1:4].transpose(1, 2, 0).reshape(n, 3 * d)], axis=-1)
    return (message, sc)


# speedup vs baseline: 1.5635x; 1.0266x over previous
import functools
import math

import jax
import jax.numpy as jnp
from jax import lax
from jax.experimental import pallas as pl
from jax.experimental.pallas import tpu as pltpu
from jax.experimental.pallas import tpu_sc as plsc

N = 10000
E = 160000
D = 128
RB = 8
HID = 64
S = 10
AVG = 16.0

NODE_BLK = 2000
EDGE_BLK = 4480

NTEC = 16
CHUNK = 48
CPI = 10
E_P = 161280
EPT_P = E_P // NTEC
ITERS = EPT_P // (CPI * CHUNK)
HALF = N // 2
GARBAGE = HALF
AGG_ROWS = 5120
ROWS_PER_TEC = AGG_ROWS // NTEC
CH = D // 2


def _node_kernel(x_ref, oh_ref, wup_ref, wskip_ref, feats_ref, sc_ref):
    x = x_ref[...]
    scale = 1.0 / math.sqrt(float(D))
    f = jnp.dot(x, wup_ref[...], preferred_element_type=jnp.float32) * scale
    feats_ref[0] = f[:, :CH]
    feats_ref[1] = f[:, CH:]
    acc = jnp.zeros_like(x)
    for s in range(S):
        xs = x * oh_ref[:, s][:, None]
        acc = acc + jnp.dot(xs, wskip_ref[s], preferred_element_type=jnp.float32)
    sc_ref[...] = acc * scale


def _silu(x):
    return x * jax.nn.sigmoid(x)


def _mlp_kernel(rb_ref, w1_ref, w2_ref, w3_ref, rw_ref):
    h = _silu(jnp.dot(rb_ref[...], w1_ref[...], preferred_element_type=jnp.float32)
              * (1.0 / math.sqrt(float(RB))))
    h = _silu(jnp.dot(h, w2_ref[...], preferred_element_type=jnp.float32)
              * (1.0 / math.sqrt(float(HID))))
    rw = (jnp.dot(h, w3_ref[...], preferred_element_type=jnp.float32)
          * (1.0 / math.sqrt(float(HID))))
    rw_ref[0, :, :CH] = rw[:, 0:CH]
    rw_ref[0, :, CH:] = rw[:, D:D + CH]
    rw_ref[1, :, :CH] = rw[:, CH:D]
    rw_ref[1, :, CH:] = rw[:, D + CH:2 * D]


def _edge_sc_kernel(sh_hbm, send_hbm, recv_hbm, feats_hbm, rw_hbm, agg_hbm,
                    send_v, recv_v, sh_v, idx_v, msg_v, rw_v, pay_v, zero_v,
                    agg_sh, meta_sem, in_sem0, in_sem1, sc_sem0, sc_sem1):
    core = lax.axis_index("c")
    sid = lax.axis_index("s")
    in_sems = (in_sem0, in_sem1)
    sc_sems = (sc_sem0, sc_sem1)

    def _zrow(r, _):
        for v in range(16):
            zero_v[r, pl.ds(v * 16, 16)] = jnp.zeros((16,), jnp.float32)
        return 0
    lax.fori_loop(0, 8, _zrow, 0)
    plsc.subcore_barrier()

    for p in range(2):
        lo = p * HALF

        zbase = sid * ROWS_PER_TEC

        def _zcopy(i, _):
            pltpu.sync_copy(zero_v, agg_sh.at[pl.ds(zbase + i * 8, 8)])
            return 0
        lax.fori_loop(0, ROWS_PER_TEC // 8, _zcopy, 0)
        plsc.subcore_barrier()

        def iter_body(t, _):
            ibase = sid * EPT_P + t * (CPI * CHUNK)
            h1 = pltpu.async_copy(send_hbm.at[pl.ds(ibase, CPI * CHUNK)],
                                  send_v, meta_sem)
            h2 = pltpu.async_copy(recv_hbm.at[pl.ds(ibase, CPI * CHUNK)],
                                  recv_v, meta_sem)
            h3 = pltpu.async_copy(sh_hbm.at[:, pl.ds(ibase, CPI * CHUNK)],
                                  sh_v, meta_sem)
            h1.wait()
            h2.wait()
            h3.wait()

            def fire_inputs(i):
                b = i % 2
                cb = ibase + i * CHUNK
                hg = pltpu.async_copy(
                    feats_hbm.at[core].at[send_v.at[pl.ds(i * CHUNK, CHUNK)]],
                    msg_v.at[b], in_sems[b])
                hr = pltpu.async_copy(rw_hbm.at[core, pl.ds(cb, CHUNK), :],
                                      rw_v.at[b], in_sems[b])
                return hg, hr

            h_in = [fire_inputs(0), fire_inputs(1)] + [None] * (CPI - 2)
            h_sc = [None] * CPI
            row_ids = [jnp.full((16,), j, jnp.int32) for j in range(4)]
            for i in range(CPI):
                b = i % 2
                hg, hr = h_in[i]
                hg.wait()
                hr.wait()
                if i >= 2:
                    h_sc[i - 2].wait()

                @plsc.parallel_loop(0, CHUNK // 16, unroll=3)
                def idx_body(g):
                    rv = recv_v[pl.ds(i * CHUNK + g * 16, 16)]
                    valid = (rv >= lo) & (rv < lo + HALF)
                    idx_v[b, pl.ds(g * 16, 16)] = jnp.where(valid, rv - lo,
                                                            GARBAGE)

                @plsc.parallel_loop(0, CHUNK, unroll=4)
                def edge_body(e):
                    es = jnp.full((16,), i * CHUNK + e, jnp.int32)
                    s0 = plsc.load_gather(sh_v, [row_ids[0], es])
                    s1 = plsc.load_gather(sh_v, [row_ids[1], es])
                    s2 = plsc.load_gather(sh_v, [row_ids[2], es])
                    s3 = plsc.load_gather(sh_v, [row_ids[3], es])
                    for v in range(CH // 16):
                        m = msg_v[b, e, pl.ds(v * 16, 16)]
                        w0 = rw_v[b, e, pl.ds(v * 16, 16)]
                        w1 = rw_v[b, e, pl.ds(CH + v * 16, 16)]
                        p0 = m * w0
                        p1 = m * w1
                        pay_v[b, e, pl.ds(v * 16, 16)] = p0 * s0
                        pay_v[b, e, pl.ds(CH + v * 16, 16)] = p1 * s1
                        pay_v[b, e, pl.ds(2 * CH + v * 16, 16)] = p1 * s2
                        pay_v[b, e, pl.ds(3 * CH + v * 16, 16)] = p1 * s3

                h_sc[i] = pltpu.async_copy(pay_v.at[b],
                                           agg_sh.at[idx_v.at[b]],
                                           sc_sems[b], add=True)
                if i + 2 < CPI:
                    h_in[i + 2] = fire_inputs(i + 2)
            h_sc[CPI - 2].wait()
            h_sc[CPI - 1].wait()
            return 0
        lax.fori_loop(0, ITERS, iter_body, 0)
        plsc.subcore_barrier()

        obase = sid * 312
        for off in (0, 104, 208):
            pltpu.sync_copy(agg_sh.at[pl.ds(obase + off, 104)],
                            agg_hbm.at[core, pl.ds(lo + obase + off, 104)])

        @pl.when(sid == NTEC - 1)
        def _():
            pltpu.sync_copy(agg_sh.at[pl.ds(4992, 8)],
                            agg_hbm.at[core, pl.ds(lo + 4992, 8)])
        plsc.subcore_barrier()


def _mix_kernel(agg_ref, w0_ref, w1_ref, out_ref):
    scale = 1.0 / (math.sqrt(float(D)) * AVG)
    a0 = agg_ref[0]
    a1 = agg_ref[1]
    for j in range(4):
        w = w0_ref[...] if j == 0 else w1_ref[...]
        o = (jnp.dot(a0[:, j * CH:(j + 1) * CH], w[:CH, :],
                     preferred_element_type=jnp.float32)
             + jnp.dot(a1[:, j * CH:(j + 1) * CH], w[CH:, :],
                       preferred_element_type=jnp.float32))
        out_ref[j] = o * scale


def kernel(node_feats, species, sh, radial_basis, senders, receivers,
           W_up, W_mlp1, W_mlp2, W_mlp3, W_lin0, W_lin1, W_skip):
    n, d = node_feats.shape

    onehot = jax.nn.one_hot(species, S, dtype=jnp.float32)

    feats_sc, sc = pl.pallas_call(
        _node_kernel,
        grid=(n // NODE_BLK,),
        in_specs=[
            pl.BlockSpec((NODE_BLK, d), lambda i: (i, 0)),
            pl.BlockSpec((NODE_BLK, S), lambda i: (i, 0)),
            pl.BlockSpec((d, d), lambda i: (0, 0)),
            pl.BlockSpec((S, d, d), lambda i: (0, 0, 0)),
        ],
        out_specs=[
            pl.BlockSpec((2, NODE_BLK, CH), lambda i: (0, i, 0)),
            pl.BlockSpec((NODE_BLK, d), lambda i: (i, 0)),
        ],
        out_shape=[
            jax.ShapeDtypeStruct((2, n, CH), jnp.float32),
            jax.ShapeDtypeStruct((n, d), jnp.float32),
        ],
    )(node_feats, onehot, W_up, W_skip)

    pad = E_P - E
    rb_p = jnp.pad(radial_basis, ((0, pad), (0, 0)))
    send_p = jnp.pad(senders, (0, pad))
    recv_p = jnp.pad(receivers, (0, pad), constant_values=N)
    shT_p = jnp.pad(sh.T, ((0, 0), (0, pad)))

    rw_sc = pl.pallas_call(
        _mlp_kernel,
        grid=(E_P // EDGE_BLK,),
        in_specs=[
            pl.BlockSpec((EDGE_BLK, RB), lambda i: (i, 0)),
            pl.BlockSpec((RB, HID), lambda i: (0, 0)),
            pl.BlockSpec((HID, HID), lambda i: (0, 0)),
            pl.BlockSpec((HID, 2 * D), lambda i: (0, 0)),
        ],
        out_specs=pl.BlockSpec((2, EDGE_BLK, d), lambda i: (0, i, 0)),
        out_shape=jax.ShapeDtypeStruct((2, E_P, d), jnp.float32),
    )(rb_p, W_mlp1, W_mlp2, W_mlp3)

    mesh = plsc.VectorSubcoreMesh(core_axis_name="c", subcore_axis_name="s")
    agg = pl.kernel(
        _edge_sc_kernel,
        out_type=jax.ShapeDtypeStruct((2, n, 4 * CH), jnp.float32),
        mesh=mesh,
        scratch_types=[
            pltpu.VMEM((CPI * CHUNK,), jnp.int32),
            pltpu.VMEM((CPI * CHUNK,), jnp.int32),
            pltpu.VMEM((4, CPI * CHUNK), jnp.float32),
            pltpu.VMEM((2, CHUNK), jnp.int32),
            pltpu.VMEM((2, CHUNK, CH), jnp.float32),
            pltpu.VMEM((2, CHUNK, 2 * CH), jnp.float32),
            pltpu.VMEM((2, CHUNK, 4 * CH), jnp.float32),
            pltpu.VMEM((8, 4 * CH), jnp.float32),
            pltpu.VMEM_SHARED((AGG_ROWS, 4 * CH), jnp.float32),
            pltpu.SemaphoreType.DMA,
            pltpu.SemaphoreType.DMA,
            pltpu.SemaphoreType.DMA,
            pltpu.SemaphoreType.DMA,
            pltpu.SemaphoreType.DMA,
        ],
        compiler_params=pltpu.CompilerParams(use_tc_tiling_on_sc=False,
                                             needs_layout_passes=False),
    )(shT_p, send_p, recv_p, feats_sc, rw_sc)

    W_lins = jnp.stack([W_lin0, W_lin1], axis=0)
    out = pl.pallas_call(
        _mix_kernel,
        grid=(n // NODE_BLK,),
        in_specs=[
            pl.BlockSpec((2, NODE_BLK, 4 * CH), lambda i: (0, i, 0)),
            pl.BlockSpec((d, d), lambda i: (0, 0)),
            pl.BlockSpec((d, d), lambda i: (0, 0)),
        ],
        out_specs=pl.BlockSpec((4, NODE_BLK, d), lambda i: (0, i, 0)),
        out_shape=jax.ShapeDtypeStruct((4, n, d), jnp.float32),
    )(agg, W_lin0, W_lin1)

    message = jnp.concatenate(
        [out[0], out[1:4].transpose(1, 2, 0).reshape(n, 3 * d)], axis=-1)
    return (message, sc)
